# baseline (speedup 1.0000x reference)
import jax
import jax.numpy as jnp
from jax import lax
from jax.experimental import pallas as pl
from jax.experimental.pallas import tpu as pltpu
from jax.experimental.pallas import tpu_sc as plsc

BB = 16384
LL = 200
LP = 208
NV = LP // 16
NC, NS = 2, 16
NW = NC * NS
RPW = BB // NW
CB = 8
NPAIR = RPW // (2 * CB)
GPC = CB * LP // 128
ZITEM = 100000
ZGEN = 32
ZPOS = LP
WIDE = 128

G0, A0, O0, Y0, M0, H0, W0 = 0, 4, 20, 52, 92, 105, 130
WACT, BACT = 138, 139

_IOTA = lambda: jnp.arange(16, dtype=jnp.int32)


def _bcast(v, j):
    return v.at[jnp.full((16,), j, jnp.int32)].get(mode="promise_in_bounds")


def _sc_body(hist2, gid2, uid, zipc, sidx, act,
             item_t, genre_t, pos_t, ut128, zt128, small_t,
             out,
             small_v, genre_v, pos_v, uidw_v, zidw_v,
             hist_v, gidv_v, ivp0_v, ivp1_v, gvp0_v, gvp1_v, plv0_v, plv1_v,
             itemb0_v, itemb1_v, rowb_v, ubuf_v, zbuf_v, six_cv, act_cv,
             cnt_s, sem0, sem1, usem):
    wid = lax.axis_index("c") * NS + lax.axis_index("s")
    base = wid * RPW

    pltpu.sync_copy(small_t, small_v)
    pltpu.sync_copy(genre_t, genre_v)
    pltpu.sync_copy(pos_t, pos_v)
    pltpu.sync_copy(uid.at[pl.ds(base // 128, RPW // 128)], uidw_v)
    pltpu.sync_copy(zipc.at[pl.ds(base // 128, RPW // 128)], zidw_v)

    def preproc(rowbase, ivp_ref, gvp_ref, plv_ref, sub):
        row0 = rowbase * LP // 128
        pltpu.sync_copy(hist2.at[pl.ds(row0, GPC)], hist_v)
        pltpu.sync_copy(gid2.at[pl.ds(row0, GPC)], gidv_v)

        def prow(r, _):
            cntv = jnp.zeros((16,), jnp.int32)
            for v in range(NV):
                rv = r * NV + v
                fr = rv * 16
                iv = hist_v[fr // 128, pl.ds(fr % 128, 16)]
                gv = gidv_v[fr // 128, pl.ds(fr % 128, 16)]
                m = iv != 0
                cntv = cntv + plsc.all_reduce_population_count(m)
                f = rv * 16
                ivp_ref[f // 128, pl.ds(f % 128, 16)] = jnp.where(m, iv, ZITEM)
                gvp_ref[rv, :] = jnp.where(m, gv, ZGEN)
                plv_ref[rv, :] = jnp.where(m, _IOTA() + (v * 16), ZPOS)
            cnt_s[sub * CB + r] = jnp.maximum(cntv[0].astype(jnp.float32), 1.0)
            return 0

        lax.fori_loop(0, CB, prow, 0)

    def issue(ivp_ref, itemb_ref, sem):
        def one(j, _):
            pltpu.async_copy(item_t.at[ivp_ref.at[j]],
                             itemb_ref.at[pl.ds(j * 128, 128)], sem)
            return 0
        lax.fori_loop(0, GPC, one, 0)

    def drain(itemb_ref, sem):
        pltpu.make_async_copy(item_t.at[pl.ds(0, CB * LP)], itemb_ref,
                              sem).wait()

    wact_row = small_v[WACT, :]
    bact_row = small_v[BACT, :]

    def pair_body(i, _):
        pairbase = base + i * 16

        for k in range(7):
            flat = k * BB + pairbase
            pltpu.sync_copy(sidx.at[flat // 128, pl.ds(flat % 128, 16)],
                            six_cv.at[k])
        pltpu.sync_copy(act.at[pl.ds(pairbase, 16)], act_cv)
        uvec = uidw_v[i // 8, pl.ds((i % 8) * 16, 16)]
        zvec = zidw_v[i // 8, pl.ds((i % 8) * 16, 16)]
        du = pltpu.async_copy(ut128.at[jnp.right_shift(uvec, 3)], ubuf_v, usem)
        dz = pltpu.async_copy(zt128.at[jnp.right_shift(zvec, 3)], zbuf_v, usem)

        preproc(pairbase + CB, ivp1_v, gvp1_v, plv1_v, 1)
        issue(ivp1_v, itemb1_v, sem1)

        sv = [six_cv[k, :] for k in range(7)]
        avec = act_cv[:]
        du.wait()
        dz.wait()

        def assemble(r, itemb_ref, gvp_ref, plv_ref):
            rl = r % CB

            def vbody(v, acc):
                rv = rl * NV + v
                gvec = gvp_ref[rv, :]
                pvec = plv_ref[rv, :]
                for j in range(16):
                    acc = (acc + itemb_ref[rv * 16 + j, :]
                           + plsc.load_gather(genre_v, [_bcast(gvec, j),
                                                        _IOTA()])
                           + plsc.load_gather(pos_v, [_bcast(pvec, j),
                                                      _IOTA()]))
                return acc

            acc = lax.fori_loop(0, NV, vbody, jnp.zeros((16,), jnp.float32))
            seq = acc / cnt_s[r]

            urow = plsc.load_gather(
                ubuf_v, [jnp.full((16,), r, jnp.int32),
                         (_bcast(uvec, r) & 7) * 16 + _IOTA()])
            zrow = plsc.load_gather(
                zbuf_v, [jnp.full((16,), r, jnp.int32),
                         (_bcast(zvec, r) & 7) * 16 + _IOTA()])
            srow = [plsc.load_gather(small_v, [_bcast(sv[k], r) + off,
                                               _IOTA()])
                    for k, off in enumerate((G0, A0, O0, Y0, M0, H0, W0))]
            arow = jnp.maximum(_bcast_f(avec, r) * wact_row + bact_row, 0.0)

            rowb_v[r, pl.ds(0, 16)] = urow
            rowb_v[r, pl.ds(16, 16)] = srow[0]
            rowb_v[r, pl.ds(24, 16)] = srow[1]
            rowb_v[r, pl.ds(32, 16)] = srow[2]
            rowb_v[r, pl.ds(40, 16)] = zrow
            rowb_v[r, pl.ds(56, 16)] = srow[3]
            rowb_v[r, pl.ds(64, 16)] = srow[4]
            rowb_v[r, pl.ds(72, 16)] = srow[5]
            rowb_v[r, pl.ds(80, 16)] = srow[6]
            rowb_v[r, pl.ds(88, 16)] = arow
            rowb_v[r, pl.ds(112, 16)] = jnp.zeros((16,), jnp.float32)
            rowb_v[r, pl.ds(104, 16)] = seq

        drain(itemb0_v, sem0)
        for r in range(CB):
            assemble(r, itemb0_v, gvp0_v, plv0_v)

        nxt = jnp.minimum(pairbase + 2 * CB, base + RPW - 2 * CB)
        preproc(nxt, ivp0_v, gvp0_v, plv0_v, 0)
        issue(ivp0_v, itemb0_v, sem0)

        drain(itemb1_v, sem1)
        for r in range(CB, 2 * CB):
            assemble(r, itemb1_v, gvp1_v, plv1_v)

        pltpu.sync_copy(rowb_v, out.at[pl.ds(pairbase, 16)])
        return 0

    preproc(base, ivp0_v, gvp0_v, plv0_v, 0)
    issue(ivp0_v, itemb0_v, sem0)
    lax.fori_loop(0, NPAIR, pair_body, 0)
    drain(itemb0_v, sem0)


def _bcast_f(v, j):
    return v.at[jnp.full((16,), j, jnp.int32)].get(mode="promise_in_bounds")


def _mlp_body(x_ref, w1_ref, b1_ref, w2_ref, b2_ref, w3_ref, b3_ref, o_ref):
    h = jnp.dot(x_ref[:], w1_ref[:], preferred_element_type=jnp.float32)
    h = jnp.maximum(h + b1_ref[:], 0.0)
    h = jnp.dot(h, w2_ref[:], preferred_element_type=jnp.float32)
    h = jnp.maximum(h + b2_ref[:], 0.0)
    o = jnp.dot(h, w3_ref[:], preferred_element_type=jnp.float32)
    o_ref[:] = o + b3_ref[:]


def kernel(user_id, gender, age, occup, zip_code, year, month, hour, weekday,
           user_activity, hist_movie_ids, hist_genre_ids,
           item_table, genre_table, pos_table, user_table, gender_table,
           age_table, occup_table, zip_table, year_table, month_table,
           weekday_table, hour_table, W_act, b_act, W1, b1, W2, b2, W3, b3):
    f32 = jnp.float32
    i32 = jnp.int32

    hist2 = jnp.pad(hist_movie_ids.astype(i32),
                    ((0, 0), (0, LP - LL))).reshape(BB * LP // 128, 128)
    gid2 = jnp.pad(hist_genre_ids.astype(i32),
                   ((0, 0), (0, LP - LL))).reshape(BB * LP // 128, 128)
    uid2 = user_id.astype(i32).reshape(BB // 128, 128)
    zip2 = zip_code.astype(i32).reshape(BB // 128, 128)
    sidx = jnp.stack([gender, age, occup, year, month, hour, weekday]
                     ).astype(i32).reshape(7 * BB // 128, 128)

    zrow = jnp.zeros((1, 16), f32)
    item_aug = jnp.concatenate([item_table, jnp.zeros((16, 16), f32)], axis=0)
    genre_aug = jnp.concatenate([genre_table, zrow], axis=0)
    pos_aug = jnp.concatenate(
        [pos_table, jnp.zeros((LP + 1 - LL, 16), f32)], axis=0)
    ut128 = user_table.reshape(-1, 128)
    zt128 = zip_table.reshape(-1, 128)

    pad8 = lambda t: jnp.pad(t, ((0, 0), (0, 8)))
    small_t = jnp.concatenate([
        pad8(gender_table), pad8(age_table), pad8(occup_table),
        pad8(year_table), pad8(month_table), pad8(hour_table),
        pad8(weekday_table), W_act, b_act.reshape(1, 16)], axis=0)

    mesh = plsc.VectorSubcoreMesh(core_axis_name="c", subcore_axis_name="s",
                                  num_cores=NC, num_subcores=NS)
    wide = pl.kernel(
        _sc_body,
        out_type=jax.ShapeDtypeStruct((BB, WIDE), f32),
        mesh=mesh,
        compiler_params=pltpu.CompilerParams(needs_layout_passes=False,
                                             use_tc_tiling_on_sc=False),
        scratch_types=[
            pltpu.VMEM((140, 16), f32),
            pltpu.VMEM((33, 16), f32),
            pltpu.VMEM((LP + 1, 16), f32),
            pltpu.VMEM((RPW // 128, 128), i32),
            pltpu.VMEM((RPW // 128, 128), i32),
            pltpu.VMEM((GPC, 128), i32),
            pltpu.VMEM((GPC, 128), i32),
            pltpu.VMEM((GPC, 128), i32),
            pltpu.VMEM((GPC, 128), i32),
            pltpu.VMEM((CB * NV, 16), i32),
            pltpu.VMEM((CB * NV, 16), i32),
            pltpu.VMEM((CB * NV, 16), i32),
            pltpu.VMEM((CB * NV, 16), i32),
            pltpu.VMEM((CB * LP, 16), f32),
            pltpu.VMEM((CB * LP, 16), f32),
            pltpu.VMEM((2 * CB, WIDE), f32),
            pltpu.VMEM((16, 128), f32),
            pltpu.VMEM((16, 128), f32),
            pltpu.VMEM((7, 16), i32),
            pltpu.VMEM((16,), f32),
            pltpu.SMEM((2 * CB,), f32),
            pltpu.SemaphoreType.DMA,
            pltpu.SemaphoreType.DMA,
            pltpu.SemaphoreType.DMA,
        ],
    )(hist2, gid2, uid2, zip2, sidx, user_activity.astype(f32),
      item_aug, genre_aug, pos_aug, ut128, zt128, small_t)

    w1w = jnp.concatenate([W1, jnp.zeros((WIDE - 120, 256), f32)], axis=0)

    bt = 2048
    out = pl.pallas_call(
        _mlp_body,
        grid=(BB // bt,),
        in_specs=[
            pl.BlockSpec((bt, WIDE), lambda i: (i, 0)),
            pl.BlockSpec((WIDE, 256), lambda i: (0, 0)),
            pl.BlockSpec((1, 256), lambda i: (0, 0)),
            pl.BlockSpec((256, 128), lambda i: (0, 0)),
            pl.BlockSpec((1, 128), lambda i: (0, 0)),
            pl.BlockSpec((128, 64), lambda i: (0, 0)),
            pl.BlockSpec((1, 64), lambda i: (0, 0)),
        ],
        out_specs=pl.BlockSpec((bt, 64), lambda i: (i, 0)),
        out_shape=jax.ShapeDtypeStruct((BB, 64), f32),
    )(wide, w1w, b1.reshape(1, 256), W2, b2.reshape(1, 128),
      W3, b3.reshape(1, 64))
    return out


# speedup vs baseline: 1.0049x; 1.0049x over previous
import jax
import jax.numpy as jnp
from jax import lax
from jax.experimental import pallas as pl
from jax.experimental.pallas import tpu as pltpu
from jax.experimental.pallas import tpu_sc as plsc

BB = 16384
LL = 200
LP = 208
NV = LP // 16
NC, NS = 2, 16
NW = NC * NS
RPW = BB // NW
CB = 8
NPAIR = RPW // (2 * CB)
GPC = CB * LP // 128
ZITEM = 100000
ZGEN = 32
ZPOS = LP
WIDE = 128

G0, A0, O0, Y0, M0, H0, W0 = 0, 4, 20, 52, 92, 105, 130
WACT, BACT = 138, 139

_IOTA = lambda: jnp.arange(16, dtype=jnp.int32)


def _bcast(v, j):
    return v.at[jnp.full((16,), j, jnp.int32)].get(mode="promise_in_bounds")


def _sc_body(hist2, gid2, uid, zipc, sidx, act,
             item_t, genre_t, pos_t, ut128, zt128, small_t,
             out,
             small_v, genre_v, pos_v, uidw_v, zidw_v,
             hist_v, gidv_v, ivp0_v, ivp1_v, gvp0_v, gvp1_v, plv0_v, plv1_v,
             itemb0_v, itemb1_v, rowb_v, ubuf_v, zbuf_v, six_cv, act_cv,
             cnt_s, sem0, sem1, usem):
    wid = lax.axis_index("c") * NS + lax.axis_index("s")
    base = wid * RPW

    pltpu.sync_copy(small_t, small_v)
    pltpu.sync_copy(genre_t, genre_v)
    pltpu.sync_copy(pos_t, pos_v)
    pltpu.sync_copy(uid.at[pl.ds(base // 128, RPW // 128)], uidw_v)
    pltpu.sync_copy(zipc.at[pl.ds(base // 128, RPW // 128)], zidw_v)

    def preproc(rowbase, ivp_ref, gvp_ref, plv_ref, sub):
        row0 = rowbase * LP // 128
        pltpu.sync_copy(hist2.at[pl.ds(row0, GPC)], hist_v)
        pltpu.sync_copy(gid2.at[pl.ds(row0, GPC)], gidv_v)

        def prow(r, _):
            cntv = jnp.zeros((16,), jnp.int32)
            for v in range(NV):
                rv = r * NV + v
                fr = rv * 16
                iv = hist_v[fr // 128, pl.ds(fr % 128, 16)]
                gv = gidv_v[fr // 128, pl.ds(fr % 128, 16)]
                m = iv != 0
                cntv = cntv + plsc.all_reduce_population_count(m)
                f = rv * 16
                ivp_ref[f // 128, pl.ds(f % 128, 16)] = jnp.where(m, iv, ZITEM)
                gvp_ref[rv, :] = jnp.where(m, gv, ZGEN)
                plv_ref[rv, :] = jnp.where(m, _IOTA() + (v * 16), ZPOS)
            cnt_s[sub * CB + r] = jnp.maximum(cntv[0].astype(jnp.float32), 1.0)
            return 0

        lax.fori_loop(0, CB, prow, 0)

    def issue(ivp_ref, itemb_ref, sem):
        def one(j, _):
            pltpu.async_copy(item_t.at[ivp_ref.at[j]],
                             itemb_ref.at[pl.ds(j * 128, 128)], sem)
            return 0
        lax.fori_loop(0, GPC, one, 0)

    def drain(itemb_ref, sem):
        pltpu.make_async_copy(item_t.at[pl.ds(0, CB * LP)], itemb_ref,
                              sem).wait()

    wact_row = small_v[WACT, :]
    bact_row = small_v[BACT, :]

    def pair_body(i, _):
        pairbase = base + i * 16

        for k in range(7):
            flat = k * BB + pairbase
            pltpu.sync_copy(sidx.at[flat // 128, pl.ds(flat % 128, 16)],
                            six_cv.at[k])
        pltpu.sync_copy(act.at[pl.ds(pairbase, 16)], act_cv)
        uvec = uidw_v[i // 8, pl.ds((i % 8) * 16, 16)]
        zvec = zidw_v[i // 8, pl.ds((i % 8) * 16, 16)]
        du = pltpu.async_copy(ut128.at[jnp.right_shift(uvec, 3)], ubuf_v, usem)
        dz = pltpu.async_copy(zt128.at[jnp.right_shift(zvec, 3)], zbuf_v, usem)

        preproc(pairbase, ivp0_v, gvp0_v, plv0_v, 0)
        issue(ivp0_v, itemb0_v, sem0)
        preproc(pairbase + CB, ivp1_v, gvp1_v, plv1_v, 1)
        issue(ivp1_v, itemb1_v, sem1)

        sv = [six_cv[k, :] for k in range(7)]
        avec = act_cv[:]
        du.wait()
        dz.wait()

        def assemble(r, itemb_ref, gvp_ref, plv_ref):
            rl = r % CB

            def vbody(v, acc):
                rv = rl * NV + v
                gvec = gvp_ref[rv, :]
                pvec = plv_ref[rv, :]
                for j in range(16):
                    acc = (acc + itemb_ref[rv * 16 + j, :]
                           + plsc.load_gather(genre_v, [_bcast(gvec, j),
                                                        _IOTA()])
                           + plsc.load_gather(pos_v, [_bcast(pvec, j),
                                                      _IOTA()]))
                return acc

            acc = lax.fori_loop(0, NV, vbody, jnp.zeros((16,), jnp.float32))
            seq = acc / cnt_s[r]

            urow = plsc.load_gather(
                ubuf_v, [jnp.full((16,), r, jnp.int32),
                         (_bcast(uvec, r) & 7) * 16 + _IOTA()])
            zrow = plsc.load_gather(
                zbuf_v, [jnp.full((16,), r, jnp.int32),
                         (_bcast(zvec, r) & 7) * 16 + _IOTA()])
            srow = [plsc.load_gather(small_v, [_bcast(sv[k], r) + off,
                                               _IOTA()])
                    for k, off in enumerate((G0, A0, O0, Y0, M0, H0, W0))]
            arow = jnp.maximum(_bcast_f(avec, r) * wact_row + bact_row, 0.0)

            rowb_v[r, pl.ds(0, 16)] = urow
            rowb_v[r, pl.ds(16, 16)] = srow[0]
            rowb_v[r, pl.ds(24, 16)] = srow[1]
            rowb_v[r, pl.ds(32, 16)] = srow[2]
            rowb_v[r, pl.ds(40, 16)] = zrow
            rowb_v[r, pl.ds(56, 16)] = srow[3]
            rowb_v[r, pl.ds(64, 16)] = srow[4]
            rowb_v[r, pl.ds(72, 16)] = srow[5]
            rowb_v[r, pl.ds(80, 16)] = srow[6]
            rowb_v[r, pl.ds(88, 16)] = arow
            rowb_v[r, pl.ds(112, 16)] = jnp.zeros((16,), jnp.float32)
            rowb_v[r, pl.ds(104, 16)] = seq

        drain(itemb0_v, sem0)
        for r in range(CB):
            assemble(r, itemb0_v, gvp0_v, plv0_v)
        drain(itemb1_v, sem1)
        for r in range(CB, 2 * CB):
            assemble(r, itemb1_v, gvp1_v, plv1_v)

        pltpu.sync_copy(rowb_v, out.at[pl.ds(pairbase, 16)])
        return 0

    lax.fori_loop(0, NPAIR, pair_body, 0)


def _bcast_f(v, j):
    return v.at[jnp.full((16,), j, jnp.int32)].get(mode="promise_in_bounds")


def _mlp_body(x_ref, w1_ref, b1_ref, w2_ref, b2_ref, w3_ref, b3_ref, o_ref):
    h = jnp.dot(x_ref[:], w1_ref[:], preferred_element_type=jnp.float32)
    h = jnp.maximum(h + b1_ref[:], 0.0)
    h = jnp.dot(h, w2_ref[:], preferred_element_type=jnp.float32)
    h = jnp.maximum(h + b2_ref[:], 0.0)
    o = jnp.dot(h, w3_ref[:], preferred_element_type=jnp.float32)
    o_ref[:] = o + b3_ref[:]


def kernel(user_id, gender, age, occup, zip_code, year, month, hour, weekday,
           user_activity, hist_movie_ids, hist_genre_ids,
           item_table, genre_table, pos_table, user_table, gender_table,
           age_table, occup_table, zip_table, year_table, month_table,
           weekday_table, hour_table, W_act, b_act, W1, b1, W2, b2, W3, b3):
    f32 = jnp.float32
    i32 = jnp.int32

    hist2 = jnp.pad(hist_movie_ids.astype(i32),
                    ((0, 0), (0, LP - LL))).reshape(BB * LP // 128, 128)
    gid2 = jnp.pad(hist_genre_ids.astype(i32),
                   ((0, 0), (0, LP - LL))).reshape(BB * LP // 128, 128)
    uid2 = user_id.astype(i32).reshape(BB // 128, 128)
    zip2 = zip_code.astype(i32).reshape(BB // 128, 128)
    sidx = jnp.stack([gender, age, occup, year, month, hour, weekday]
                     ).astype(i32).reshape(7 * BB // 128, 128)

    zrow = jnp.zeros((1, 16), f32)
    item_aug = jnp.concatenate([item_table, jnp.zeros((16, 16), f32)], axis=0)
    genre_aug = jnp.concatenate([genre_table, zrow], axis=0)
    pos_aug = jnp.concatenate(
        [pos_table, jnp.zeros((LP + 1 - LL, 16), f32)], axis=0)
    ut128 = user_table.reshape(-1, 128)
    zt128 = zip_table.reshape(-1, 128)

    pad8 = lambda t: jnp.pad(t, ((0, 0), (0, 8)))
    small_t = jnp.concatenate([
        pad8(gender_table), pad8(age_table), pad8(occup_table),
        pad8(year_table), pad8(month_table), pad8(hour_table),
        pad8(weekday_table), W_act, b_act.reshape(1, 16)], axis=0)

    mesh = plsc.VectorSubcoreMesh(core_axis_name="c", subcore_axis_name="s",
                                  num_cores=NC, num_subcores=NS)
    wide = pl.kernel(
        _sc_body,
        out_type=jax.ShapeDtypeStruct((BB, WIDE), f32),
        mesh=mesh,
        compiler_params=pltpu.CompilerParams(needs_layout_passes=False,
                                             use_tc_tiling_on_sc=False),
        scratch_types=[
            pltpu.VMEM((140, 16), f32),
            pltpu.VMEM((33, 16), f32),
            pltpu.VMEM((LP + 1, 16), f32),
            pltpu.VMEM((RPW // 128, 128), i32),
            pltpu.VMEM((RPW // 128, 128), i32),
            pltpu.VMEM((GPC, 128), i32),
            pltpu.VMEM((GPC, 128), i32),
            pltpu.VMEM((GPC, 128), i32),
            pltpu.VMEM((GPC, 128), i32),
            pltpu.VMEM((CB * NV, 16), i32),
            pltpu.VMEM((CB * NV, 16), i32),
            pltpu.VMEM((CB * NV, 16), i32),
            pltpu.VMEM((CB * NV, 16), i32),
            pltpu.VMEM((CB * LP, 16), f32),
            pltpu.VMEM((CB * LP, 16), f32),
            pltpu.VMEM((2 * CB, WIDE), f32),
            pltpu.VMEM((16, 128), f32),
            pltpu.VMEM((16, 128), f32),
            pltpu.VMEM((7, 16), i32),
            pltpu.VMEM((16,), f32),
            pltpu.SMEM((2 * CB,), f32),
            pltpu.SemaphoreType.DMA,
            pltpu.SemaphoreType.DMA,
            pltpu.SemaphoreType.DMA,
        ],
    )(hist2, gid2, uid2, zip2, sidx, user_activity.astype(f32),
      item_aug, genre_aug, pos_aug, ut128, zt128, small_t)

    w1w = jnp.concatenate([W1, jnp.zeros((WIDE - 120, 256), f32)], axis=0)

    bt = 2048
    out = pl.pallas_call(
        _mlp_body,
        grid=(BB // bt,),
        in_specs=[
            pl.BlockSpec((bt, WIDE), lambda i: (i, 0)),
            pl.BlockSpec((WIDE, 256), lambda i: (0, 0)),
            pl.BlockSpec((1, 256), lambda i: (0, 0)),
            pl.BlockSpec((256, 128), lambda i: (0, 0)),
            pl.BlockSpec((1, 128), lambda i: (0, 0)),
            pl.BlockSpec((128, 64), lambda i: (0, 0)),
            pl.BlockSpec((1, 64), lambda i: (0, 0)),
        ],
        out_specs=pl.BlockSpec((bt, 64), lambda i: (i, 0)),
        out_shape=jax.ShapeDtypeStruct((BB, 64), f32),
    )(wide, w1w, b1.reshape(1, 256), W2, b2.reshape(1, 128),
      W3, b3.reshape(1, 64))
    return out
